# baseline (speedup 1.0000x reference)
import functools

import jax
import jax.numpy as jnp
from jax import lax
from jax.experimental import pallas as pl
from jax.experimental.pallas import tpu as pltpu
from jax.experimental.pallas import tpu_sc as plsc

GAMMA = 12.0
HIDDEN = 128
BATCH = 16384
LANES = 16
NUM_WORKERS = 32
B_PER_W = BATCH // NUM_WORKERS
CHUNK = 128
NCHUNK = B_PER_W // CHUNK
SPLIT = 96


def _transe_body(ent_hbm, rel_hbm, head_hbm, ridx_hbm, tail_hbm, out_hbm,
                 hidx_v, ridx_v, tidx_v, out_v, pair_v,
                 h0, t0, h1, t1, semh0, semrt0, semh1, semrt1):
    wid = lax.axis_index("s") * 2 + lax.axis_index("c")
    base = wid * B_PER_W
    lane = lax.iota(jnp.int32, 16)
    lo8 = lane < 8
    lanehalf = lane >> 1
    cidx = (lane & 1) * 8

    cps = [
        pltpu.async_copy(head_hbm.at[pl.ds(base, B_PER_W)], hidx_v, semh0),
        pltpu.async_copy(ridx_hbm.at[pl.ds(base, B_PER_W)], ridx_v, semh0),
        pltpu.async_copy(tail_hbm.at[pl.ds(base, B_PER_W)], tidx_v, semh0),
    ]
    for cp in cps:
        cp.wait()

    bufs = [(h0, t0, semh0, semrt0), (h1, t1, semh1, semrt1)]

    def fire_ht(c, which):
        h_v, t_v, semh, semrt = bufs[which]
        sl = pl.ds(c * CHUNK, CHUNK)
        pltpu.async_copy(ent_hbm.at[hidx_v.at[sl]], h_v, semh)
        pltpu.async_copy(ent_hbm.at[tidx_v.at[sl]], t_v, semrt)

    def wait_h(which):
        h_v, _, semh, _ = bufs[which]
        pltpu.make_async_copy(ent_hbm.at[pl.ds(0, CHUNK)], h_v, semh).wait()

    def fire_radd(c, which):
        h_v, _, _, semrt = bufs[which]
        sl = pl.ds(c * CHUNK, CHUNK)
        pltpu.async_copy(rel_hbm.at[ridx_v.at[sl]], h_v, semrt, add=True)

    def drain_rt(which):
        h_v, t_v, _, semrt = bufs[which]
        dummy = ent_hbm.at[pl.ds(0, CHUNK)]
        pltpu.make_async_copy(dummy, t_v, semrt).wait()
        pltpu.make_async_copy(dummy, h_v, semrt).wait()

    def shuf(x, s):
        return x.at[lane ^ s].get(mode="promise_in_bounds")

    def phase_a(c, which, lo, hi):
        h_v, t_v, _, _ = bufs[which]

        @plsc.parallel_loop(lo, hi, step=2)
        def pair_rows(g):
            def row_acc(row):
                acc = None
                for k in range(HIDDEN // LANES):
                    hrv = h_v[row, pl.ds(k * 16, 16)]
                    tv = t_v[row, pl.ds(k * 16, 16)]
                    d = jnp.abs(hrv - tv)
                    acc = d if acc is None else acc + d
                return acc

            a = row_acc(g)
            b = row_acc(g + 1)
            fold = jnp.where(lo8, a + shuf(a, 8), b + shuf(b, 8))
            for s in (4, 2, 1):
                fold = fold + shuf(fold, s)
            pair_v[pl.ds(g * 8, 16)] = fold

    def phase_b(c):
        @plsc.parallel_loop(0, CHUNK // 16, step=1)
        def window(w):
            svec = jnp.zeros((16,), jnp.float32)
            for i in range(8):
                v = pair_v[pl.ds((8 * w + i) * 16, 16)]
                svec = jnp.where(lanehalf == i,
                                 v.at[cidx].get(mode="promise_in_bounds"), svec)
            out_v[pl.ds(c * CHUNK + w * 16, 16)] = GAMMA - svec

    fire_ht(0, 0)
    wait_h(0)
    fire_radd(0, 0)

    def pair(cc, carry):
        c0 = 2 * cc
        fire_ht(c0 + 1, 1)
        drain_rt(0)
        phase_a(c0, 0, 0, SPLIT)
        wait_h(1)
        fire_radd(c0 + 1, 1)
        phase_a(c0, 0, SPLIT, CHUNK)
        phase_b(c0)

        @pl.when(cc < NCHUNK // 2 - 1)
        def _():
            fire_ht(c0 + 2, 0)

        drain_rt(1)
        phase_a(c0 + 1, 1, 0, SPLIT)

        @pl.when(cc < NCHUNK // 2 - 1)
        def _():
            wait_h(0)
            fire_radd(c0 + 2, 0)

        phase_a(c0 + 1, 1, SPLIT, CHUNK)
        phase_b(c0 + 1)
        return carry

    lax.fori_loop(0, NCHUNK // 2, pair, 0)

    pltpu.sync_copy(out_v, out_hbm.at[pl.ds(base, B_PER_W)])


def kernel(entity_emb, relation_emb, head, relation, tail):
    head = head.astype(jnp.int32)
    relation = relation.astype(jnp.int32)
    tail = tail.astype(jnp.int32)
    mesh = plsc.VectorSubcoreMesh(core_axis_name="c", subcore_axis_name="s")
    f = functools.partial(
        pl.kernel,
        mesh=mesh,
        out_type=jax.ShapeDtypeStruct((BATCH,), jnp.float32),
        scratch_types=[
            pltpu.VMEM((B_PER_W,), jnp.int32),
            pltpu.VMEM((B_PER_W,), jnp.int32),
            pltpu.VMEM((B_PER_W,), jnp.int32),
            pltpu.VMEM((B_PER_W,), jnp.float32),
            pltpu.VMEM((CHUNK * 8,), jnp.float32),
            pltpu.VMEM((CHUNK, HIDDEN), jnp.float32),
            pltpu.VMEM((CHUNK, HIDDEN), jnp.float32),
            pltpu.VMEM((CHUNK, HIDDEN), jnp.float32),
            pltpu.VMEM((CHUNK, HIDDEN), jnp.float32),
            pltpu.SemaphoreType.DMA,
            pltpu.SemaphoreType.DMA,
            pltpu.SemaphoreType.DMA,
            pltpu.SemaphoreType.DMA,
        ],
    )(_transe_body)
    return f(entity_emb, relation_emb, head, relation, tail)


# speedup vs baseline: 1.0642x; 1.0642x over previous
import functools

import jax
import jax.numpy as jnp
from jax import lax
from jax.experimental import pallas as pl
from jax.experimental.pallas import tpu as pltpu
from jax.experimental.pallas import tpu_sc as plsc

GAMMA = 12.0
HIDDEN = 128
BATCH = 16384
LANES = 16
NUM_WORKERS = 32
B_PER_W = BATCH // NUM_WORKERS
CHUNK = 64
NCHUNK = B_PER_W // CHUNK


def _transe_body(ent_hbm, rel_hbm, head_hbm, ridx_hbm, tail_hbm, out_hbm,
                 hidx_v, ridx_v, tidx_v, out_v, pair_v,
                 h0, r0, t0, h1, r1, t1, sem0, sem1):
    wid = lax.axis_index("s") * 2 + lax.axis_index("c")
    base = wid * B_PER_W
    lane = lax.iota(jnp.int32, 16)
    lo8 = lane < 8
    lanehalf = lane >> 1
    cidx = (lane & 1) * 8

    cps = [
        pltpu.async_copy(head_hbm.at[pl.ds(base, B_PER_W)], hidx_v, sem0),
        pltpu.async_copy(ridx_hbm.at[pl.ds(base, B_PER_W)], ridx_v, sem0),
        pltpu.async_copy(tail_hbm.at[pl.ds(base, B_PER_W)], tidx_v, sem0),
    ]
    for cp in cps:
        cp.wait()

    bufs = [(h0, r0, t0, sem0), (h1, r1, t1, sem1)]

    def fire(c, which):
        h_v, r_v, t_v, sem = bufs[which]
        sl = pl.ds(c * CHUNK, CHUNK)
        return [
            pltpu.async_copy(ent_hbm.at[hidx_v.at[sl]], h_v, sem),
            pltpu.async_copy(rel_hbm.at[ridx_v.at[sl]], r_v, sem),
            pltpu.async_copy(ent_hbm.at[tidx_v.at[sl]], t_v, sem),
        ]

    def drain(which):
        h_v, r_v, t_v, sem = bufs[which]
        dummy = ent_hbm.at[pl.ds(0, CHUNK)]
        for dst in (h_v, r_v, t_v):
            pltpu.make_async_copy(dummy, dst, sem).wait()

    def shuf(x, s):
        return x.at[lane ^ s].get(mode="promise_in_bounds")

    def compute(c, which):
        h_v, r_v, t_v, _ = bufs[which]

        @plsc.parallel_loop(0, CHUNK, step=2)
        def pair_rows(g):
            def row_acc(row):
                acc = None
                for k in range(HIDDEN // LANES):
                    hv = h_v[row, pl.ds(k * 16, 16)]
                    rv = r_v[row, pl.ds(k * 16, 16)]
                    tv = t_v[row, pl.ds(k * 16, 16)]
                    d = jnp.abs(hv + rv - tv)
                    acc = d if acc is None else acc + d
                return acc

            a = row_acc(g)
            b = row_acc(g + 1)
            fold = jnp.where(lo8, a + shuf(a, 8), b + shuf(b, 8))
            for s in (4, 2, 1):
                fold = fold + shuf(fold, s)
            pair_v[pl.ds(g * 8, 16)] = fold

        @plsc.parallel_loop(0, CHUNK // 16, step=1)
        def window(w):
            svec = jnp.zeros((16,), jnp.float32)
            for i in range(8):
                v = pair_v[pl.ds((8 * w + i) * 16, 16)]
                svec = jnp.where(lanehalf == i,
                                 v.at[cidx].get(mode="promise_in_bounds"), svec)
            out_v[pl.ds(c * CHUNK + w * 16, 16)] = GAMMA - svec

    fire(0, 0)

    def pair(cc, carry):
        c0 = 2 * cc
        cps1 = fire(c0 + 1, 1)
        drain(0)
        compute(c0, 0)

        @pl.when(cc < NCHUNK // 2 - 1)
        def _():
            fire(c0 + 2, 0)

        for cp in cps1:
            cp.wait()
        compute(c0 + 1, 1)
        return carry

    lax.fori_loop(0, NCHUNK // 2, pair, 0)

    pltpu.sync_copy(out_v, out_hbm.at[pl.ds(base, B_PER_W)])


def kernel(entity_emb, relation_emb, head, relation, tail):
    head = head.astype(jnp.int32)
    relation = relation.astype(jnp.int32)
    tail = tail.astype(jnp.int32)
    mesh = plsc.VectorSubcoreMesh(core_axis_name="c", subcore_axis_name="s")
    f = functools.partial(
        pl.kernel,
        mesh=mesh,
        out_type=jax.ShapeDtypeStruct((BATCH,), jnp.float32),
        scratch_types=[
            pltpu.VMEM((B_PER_W,), jnp.int32),
            pltpu.VMEM((B_PER_W,), jnp.int32),
            pltpu.VMEM((B_PER_W,), jnp.int32),
            pltpu.VMEM((B_PER_W,), jnp.float32),
            pltpu.VMEM((CHUNK * 8,), jnp.float32),
            pltpu.VMEM((CHUNK, HIDDEN), jnp.float32),
            pltpu.VMEM((CHUNK, HIDDEN), jnp.float32),
            pltpu.VMEM((CHUNK, HIDDEN), jnp.float32),
            pltpu.VMEM((CHUNK, HIDDEN), jnp.float32),
            pltpu.VMEM((CHUNK, HIDDEN), jnp.float32),
            pltpu.VMEM((CHUNK, HIDDEN), jnp.float32),
            pltpu.SemaphoreType.DMA,
            pltpu.SemaphoreType.DMA,
        ],
    )(_transe_body)
    return f(entity_emb, relation_emb, head, relation, tail)
